# baseline (speedup 1.0000x reference)
import functools

import numpy as np
import jax
import jax.numpy as jnp
from jax import lax
from jax.experimental import pallas as pl
from jax.experimental.pallas import tpu as pltpu
from jax.experimental.pallas import tpu_sc as plsc

_N = 10000
_E = 320000
_HID = 128
_OUT = 64
_LAYERS = 4
_ALPHA = 0.1
_THETA = 0.5

_CHUNK = 64
_NCORES = 2
_NSUB = 16
_NW = _NCORES * _NSUB
_IDXROWS = 160
_NCHUNKS = _NW * _IDXROWS
_EPAD = _NCHUNKS * _CHUNK
_STG = 40
_NSTG = _IDXROWS // _STG
_SPAIRS = _STG // 2
_NZ = 16
_ZCHUNKS = _N // _NZ

_ROWBLK = 1000
_GRID = _N // _ROWBLK


_ZROWS = 640
_ZLAST = (_N + 128) - 15 * _ZROWS


def _sc_aggregate(h, src3d, dst3d, zrows):
    mesh = plsc.VectorSubcoreMesh(core_axis_name="c", subcore_axis_name="s")

    @functools.partial(
        pl.kernel,
        out_type=jax.ShapeDtypeStruct((_NCORES, _N, _HID), jnp.float32),
        mesh=mesh,
        scratch_types=[
            pltpu.VMEM((_STG, 1, _CHUNK), jnp.int32),
            pltpu.VMEM((_STG, 1, _CHUNK), jnp.int32),
            pltpu.VMEM((_STG, 1, _CHUNK), jnp.int32),
            pltpu.VMEM((_STG, 1, _CHUNK), jnp.int32),
            pltpu.VMEM((_CHUNK, _HID), jnp.float32),
            pltpu.VMEM((_CHUNK, _HID), jnp.float32),
            pltpu.VMEM_SHARED((_N + 128, _HID), jnp.float32),
            pltpu.SemaphoreType.DMA,
            pltpu.SemaphoreType.DMA,
            pltpu.SemaphoreType.DMA,
            pltpu.SemaphoreType.DMA,
        ],
    )
    def k(h_hbm, src_hbm, dst_hbm, z_hbm, out_hbm, sidx0, sidx1, didx0, didx1,
          rows0, rows1, agg, sem_ia, sem_ib, sem_g, sem_o):
        cid = lax.axis_index("c")
        sid = lax.axis_index("s")
        wid = cid * _NSUB + sid

        lo = wid * _IDXROWS
        sbufs = (sidx0, sidx1)
        dbufs = (didx0, didx1)
        isems = (sem_ia, sem_ib)

        for s in range(2):
            pltpu.async_copy(src_hbm.at[pl.ds(lo + s * _STG, _STG)],
                             sbufs[s], isems[s])
            pltpu.async_copy(dst_hbm.at[pl.ds(lo + s * _STG, _STG)],
                             dbufs[s], isems[s])

        @pl.when(sid < 15)
        def _():
            pltpu.async_copy(z_hbm, agg.at[pl.ds(sid * _ZROWS, _ZROWS)], sem_o)

        @pl.when(sid == 15)
        def _():
            pltpu.async_copy(z_hbm.at[pl.ds(0, _ZLAST)],
                             agg.at[pl.ds(15 * _ZROWS, _ZLAST)], sem_o)

        for s in range(2):
            pltpu.make_async_copy(src_hbm.at[pl.ds(0, _STG)], sbufs[s],
                                  isems[s]).wait()
            pltpu.make_async_copy(dst_hbm.at[pl.ds(0, _STG)], dbufs[s],
                                  isems[s]).wait()

        pltpu.async_copy(h_hbm.at[sidx0.at[0, 0]], rows0, sem_g)

        @pl.when(sid < 15)
        def _():
            pltpu.make_async_copy(z_hbm, agg.at[pl.ds(sid * _ZROWS, _ZROWS)],
                                  sem_o).wait()

        @pl.when(sid == 15)
        def _():
            pltpu.make_async_copy(z_hbm.at[pl.ds(0, _ZLAST)],
                                  agg.at[pl.ds(15 * _ZROWS, _ZLAST)],
                                  sem_o).wait()

        plsc.subcore_barrier()

        def g_wait(rbuf):
            pltpu.make_async_copy(h_hbm.at[pl.ds(0, _CHUNK)], rbuf, sem_g).wait()

        for s in range(_NSTG):
            sbuf, dbuf, isem = sbufs[s % 2], dbufs[s % 2], isems[s % 2]
            nsbuf = sbufs[(s + 1) % 2]

            @pl.loop(0, _SPAIRS)
            def _(p):
                i0 = 2 * p
                g_wait(rows0)
                pltpu.async_copy(h_hbm.at[sbuf.at[i0 + 1, 0]], rows1, sem_g)
                pltpu.sync_copy(rows0, agg.at[dbuf.at[i0, 0]], add=True)
                g_wait(rows1)

                @pl.when(p < _SPAIRS - 1)
                def _():
                    pltpu.async_copy(h_hbm.at[sbuf.at[i0 + 2, 0]], rows0, sem_g)

                if s + 1 < _NSTG:
                    @pl.when(p == _SPAIRS - 1)
                    def _():
                        pltpu.async_copy(h_hbm.at[nsbuf.at[0, 0]], rows0, sem_g)

                pltpu.sync_copy(rows1, agg.at[dbuf.at[i0 + 1, 0]], add=True)

            if s + 2 < _NSTG:
                pltpu.async_copy(src_hbm.at[pl.ds(lo + (s + 2) * _STG, _STG)],
                                 sbuf, isem)
                pltpu.async_copy(dst_hbm.at[pl.ds(lo + (s + 2) * _STG, _STG)],
                                 dbuf, isem)
                pltpu.make_async_copy(src_hbm.at[pl.ds(0, _STG)], sbuf,
                                      isem).wait()
                pltpu.make_async_copy(dst_hbm.at[pl.ds(0, _STG)], dbuf,
                                      isem).wait()

        plsc.subcore_barrier()

        @pl.loop(sid, _ZCHUNKS, step=_NSUB)
        def _(z):
            pltpu.async_copy(agg.at[pl.ds(z * _NZ, _NZ)],
                             out_hbm.at[cid, pl.ds(z * _NZ, _NZ)], sem_o)

        @pl.loop(sid, _ZCHUNKS, step=_NSUB)
        def _(z):
            pltpu.make_async_copy(agg.at[pl.ds(z * _NZ, _NZ)],
                                  out_hbm.at[cid, pl.ds(z * _NZ, _NZ)],
                                  sem_o).wait()

    return k(h, src3d, dst3d, zrows)


def _tc_entry(x, w0t, b0):
    def body(x_ref, w_ref, b_ref, o_ref):
        y = jnp.dot(x_ref[...], w_ref[...], preferred_element_type=jnp.float32)
        o_ref[...] = jnp.maximum(y + b_ref[...], 0.0)

    return pl.pallas_call(
        body,
        grid=(_GRID,),
        in_specs=[
            pl.BlockSpec((_ROWBLK, _HID), lambda i: (i, 0)),
            pl.BlockSpec((_HID, _HID), lambda i: (0, 0)),
            pl.BlockSpec((1, _HID), lambda i: (0, 0)),
        ],
        out_specs=pl.BlockSpec((_ROWBLK, _HID), lambda i: (i, 0)),
        out_shape=jax.ShapeDtypeStruct((_N, _HID), jnp.float32),
    )(x, w0t, b0)


def _tc_layer(parts, x0, wc_l, beta):
    one_m_a = 1.0 - _ALPHA
    one_m_b = 1.0 - beta

    def body(pa_ref, pb_ref, x0_ref, w_ref, o_ref):
        agg = pa_ref[0] + pb_ref[0]
        hh = one_m_a * agg + _ALPHA * x0_ref[...]
        y = jnp.dot(hh, w_ref[...], preferred_element_type=jnp.float32)
        o_ref[...] = jnp.maximum(one_m_b * hh + beta * y, 0.0)

    return pl.pallas_call(
        body,
        grid=(_GRID,),
        in_specs=[
            pl.BlockSpec((1, _ROWBLK, _HID), lambda i: (0, i, 0)),
            pl.BlockSpec((1, _ROWBLK, _HID), lambda i: (1, i, 0)),
            pl.BlockSpec((_ROWBLK, _HID), lambda i: (i, 0)),
            pl.BlockSpec((_HID, _HID), lambda i: (0, 0)),
        ],
        out_specs=pl.BlockSpec((_ROWBLK, _HID), lambda i: (i, 0)),
        out_shape=jax.ShapeDtypeStruct((_N, _HID), jnp.float32),
    )(parts, parts, x0, wc_l)


def _tc_final(h, w1t, b1):
    def body(h_ref, w_ref, b_ref, o_ref):
        y = jnp.dot(h_ref[...], w_ref[...], preferred_element_type=jnp.float32)
        y = y + b_ref[...]
        m = jnp.max(y, axis=-1, keepdims=True)
        e = jnp.exp(y - m)
        lse = jnp.log(jnp.sum(e, axis=-1, keepdims=True))
        o_ref[...] = y - m - lse

    return pl.pallas_call(
        body,
        grid=(_GRID,),
        in_specs=[
            pl.BlockSpec((_ROWBLK, _HID), lambda i: (i, 0)),
            pl.BlockSpec((_HID, _OUT), lambda i: (0, 0)),
            pl.BlockSpec((1, _OUT), lambda i: (0, 0)),
        ],
        out_specs=pl.BlockSpec((_ROWBLK, _OUT), lambda i: (i, 0)),
        out_shape=jax.ShapeDtypeStruct((_N, _OUT), jnp.float32),
    )(h, w1t, b1)


def kernel(x, edge_index, W0, b0, W1, b1, Wc):
    ei = edge_index.astype(jnp.int32)
    npad = _EPAD - _E
    pad_src = jnp.arange(npad, dtype=jnp.int32) % 128
    pad_dst = _N + (jnp.arange(npad, dtype=jnp.int32) % 128)
    src3d = jnp.concatenate(
        [ei[0], pad_src]).reshape(_NCHUNKS, 1, _CHUNK)
    dst3d = jnp.concatenate(
        [ei[1], pad_dst]).reshape(_NCHUNKS, 1, _CHUNK)

    zrows = jnp.zeros((_ZROWS, _HID), jnp.float32)

    h = _tc_entry(x, W0.T, b0.reshape(1, _HID))
    x0 = h
    for l in range(_LAYERS):
        parts = _sc_aggregate(h, src3d, dst3d, zrows)
        beta = float(np.log(_THETA / (l + 1) + 1.0))
        h = _tc_layer(parts, x0, Wc[l], beta)
    return _tc_final(h, W1.T, b1.reshape(1, _OUT))


# speedup vs baseline: 1.3300x; 1.3300x over previous
import functools

import numpy as np
import jax
import jax.numpy as jnp
from jax import lax
from jax.experimental import pallas as pl
from jax.experimental.pallas import tpu as pltpu
from jax.experimental.pallas import tpu_sc as plsc

_N = 10000
_E = 320000
_HID = 128
_OUT = 64
_LAYERS = 4
_ALPHA = 0.1
_THETA = 0.5

_CHUNK = 128
_NCORES = 2
_NSUB = 16
_NW = _NCORES * _NSUB
_IDXROWS = 80
_NCHUNKS = _NW * _IDXROWS
_EPAD = _NCHUNKS * _CHUNK
_STG = 20
_NSTG = _IDXROWS // _STG
_SPAIRS = _STG // 2
_NZ = 16
_ZCHUNKS = _N // _NZ

_ROWBLK = 1000
_GRID = _N // _ROWBLK


_ZROWS = 640
_ZLAST = (_N + 128) - 15 * _ZROWS


def _sc_aggregate(h, src3d, dst3d, zrows):
    mesh = plsc.VectorSubcoreMesh(core_axis_name="c", subcore_axis_name="s")

    @functools.partial(
        pl.kernel,
        out_type=jax.ShapeDtypeStruct((_NCORES, _N, _HID), jnp.float32),
        mesh=mesh,
        scratch_types=[
            pltpu.VMEM((_STG, 1, _CHUNK), jnp.int32),
            pltpu.VMEM((_STG, 1, _CHUNK), jnp.int32),
            pltpu.VMEM((_STG, 1, _CHUNK), jnp.int32),
            pltpu.VMEM((_STG, 1, _CHUNK), jnp.int32),
            pltpu.VMEM((_STG, 1, _CHUNK), jnp.int32),
            pltpu.VMEM((_STG, 1, _CHUNK), jnp.int32),
            pltpu.VMEM((_CHUNK, _HID), jnp.float32),
            pltpu.VMEM((_CHUNK, _HID), jnp.float32),
            pltpu.VMEM_SHARED((_N + 128, _HID), jnp.float32),
            pltpu.SemaphoreType.DMA,
            pltpu.SemaphoreType.DMA,
            pltpu.SemaphoreType.DMA,
            pltpu.SemaphoreType.DMA,
            pltpu.SemaphoreType.DMA,
        ],
    )
    def k(h_hbm, src_hbm, dst_hbm, z_hbm, out_hbm, sidx0, sidx1, sidx2,
          didx0, didx1, didx2, rows0, rows1, agg, sem_ia, sem_ib, sem_ic,
          sem_g, sem_o):
        cid = lax.axis_index("c")
        sid = lax.axis_index("s")
        wid = cid * _NSUB + sid

        lo = wid * _IDXROWS
        sbufs = (sidx0, sidx1, sidx2)
        dbufs = (didx0, didx1, didx2)
        isems = (sem_ia, sem_ib, sem_ic)

        for s in range(3):
            pltpu.async_copy(src_hbm.at[pl.ds(lo + s * _STG, _STG)],
                             sbufs[s], isems[s])
            pltpu.async_copy(dst_hbm.at[pl.ds(lo + s * _STG, _STG)],
                             dbufs[s], isems[s])

        @pl.when(sid < 15)
        def _():
            pltpu.async_copy(z_hbm, agg.at[pl.ds(sid * _ZROWS, _ZROWS)], sem_o)

        @pl.when(sid == 15)
        def _():
            pltpu.async_copy(z_hbm.at[pl.ds(0, _ZLAST)],
                             agg.at[pl.ds(15 * _ZROWS, _ZLAST)], sem_o)

        for s in range(3):
            pltpu.make_async_copy(src_hbm.at[pl.ds(0, _STG)], sbufs[s],
                                  isems[s]).wait()
            pltpu.make_async_copy(dst_hbm.at[pl.ds(0, _STG)], dbufs[s],
                                  isems[s]).wait()

        pltpu.async_copy(h_hbm.at[sidx0.at[0, 0]], rows0, sem_g)

        @pl.when(sid < 15)
        def _():
            pltpu.make_async_copy(z_hbm, agg.at[pl.ds(sid * _ZROWS, _ZROWS)],
                                  sem_o).wait()

        @pl.when(sid == 15)
        def _():
            pltpu.make_async_copy(z_hbm.at[pl.ds(0, _ZLAST)],
                                  agg.at[pl.ds(15 * _ZROWS, _ZLAST)],
                                  sem_o).wait()

        plsc.subcore_barrier()

        def g_wait(rbuf):
            pltpu.make_async_copy(h_hbm.at[pl.ds(0, _CHUNK)], rbuf, sem_g).wait()

        for s in range(_NSTG):
            sbuf, dbuf = sbufs[s % 3], dbufs[s % 3]
            nsbuf = sbufs[(s + 1) % 3]

            if 3 <= s + 1 < _NSTG:
                nisem = isems[(s + 1) % 3]
                pltpu.make_async_copy(src_hbm.at[pl.ds(0, _STG)], nsbuf,
                                      nisem).wait()
                pltpu.make_async_copy(dst_hbm.at[pl.ds(0, _STG)],
                                      dbufs[(s + 1) % 3], nisem).wait()

            @pl.loop(0, _SPAIRS)
            def _(p):
                i0 = 2 * p
                g_wait(rows0)
                pltpu.async_copy(h_hbm.at[sbuf.at[i0 + 1, 0]], rows1, sem_g)
                pltpu.sync_copy(rows0, agg.at[dbuf.at[i0, 0]], add=True)
                g_wait(rows1)

                @pl.when(p < _SPAIRS - 1)
                def _():
                    pltpu.async_copy(h_hbm.at[sbuf.at[i0 + 2, 0]], rows0, sem_g)

                if s + 1 < _NSTG:
                    @pl.when(p == _SPAIRS - 1)
                    def _():
                        pltpu.async_copy(h_hbm.at[nsbuf.at[0, 0]], rows0, sem_g)

                pltpu.sync_copy(rows1, agg.at[dbuf.at[i0 + 1, 0]], add=True)

            if s + 3 < _NSTG:
                pltpu.async_copy(src_hbm.at[pl.ds(lo + (s + 3) * _STG, _STG)],
                                 sbuf, isems[s % 3])
                pltpu.async_copy(dst_hbm.at[pl.ds(lo + (s + 3) * _STG, _STG)],
                                 dbuf, isems[s % 3])

        plsc.subcore_barrier()

        @pl.loop(sid, _ZCHUNKS, step=_NSUB)
        def _(z):
            pltpu.async_copy(agg.at[pl.ds(z * _NZ, _NZ)],
                             out_hbm.at[cid, pl.ds(z * _NZ, _NZ)], sem_o)

        @pl.loop(sid, _ZCHUNKS, step=_NSUB)
        def _(z):
            pltpu.make_async_copy(agg.at[pl.ds(z * _NZ, _NZ)],
                                  out_hbm.at[cid, pl.ds(z * _NZ, _NZ)],
                                  sem_o).wait()

    return k(h, src3d, dst3d, zrows)


def _tc_entry(x, w0t, b0):
    def body(x_ref, w_ref, b_ref, o_ref):
        y = jnp.dot(x_ref[...], w_ref[...], preferred_element_type=jnp.float32)
        o_ref[...] = jnp.maximum(y + b_ref[...], 0.0)

    return pl.pallas_call(
        body,
        grid=(_GRID,),
        in_specs=[
            pl.BlockSpec((_ROWBLK, _HID), lambda i: (i, 0)),
            pl.BlockSpec((_HID, _HID), lambda i: (0, 0)),
            pl.BlockSpec((1, _HID), lambda i: (0, 0)),
        ],
        out_specs=pl.BlockSpec((_ROWBLK, _HID), lambda i: (i, 0)),
        out_shape=jax.ShapeDtypeStruct((_N, _HID), jnp.float32),
    )(x, w0t, b0)


def _tc_layer(parts, x0, wc_l, beta):
    one_m_a = 1.0 - _ALPHA
    one_m_b = 1.0 - beta

    def body(pa_ref, pb_ref, x0_ref, w_ref, o_ref):
        agg = pa_ref[0] + pb_ref[0]
        hh = one_m_a * agg + _ALPHA * x0_ref[...]
        y = jnp.dot(hh, w_ref[...], preferred_element_type=jnp.float32)
        o_ref[...] = jnp.maximum(one_m_b * hh + beta * y, 0.0)

    return pl.pallas_call(
        body,
        grid=(_GRID,),
        in_specs=[
            pl.BlockSpec((1, _ROWBLK, _HID), lambda i: (0, i, 0)),
            pl.BlockSpec((1, _ROWBLK, _HID), lambda i: (1, i, 0)),
            pl.BlockSpec((_ROWBLK, _HID), lambda i: (i, 0)),
            pl.BlockSpec((_HID, _HID), lambda i: (0, 0)),
        ],
        out_specs=pl.BlockSpec((_ROWBLK, _HID), lambda i: (i, 0)),
        out_shape=jax.ShapeDtypeStruct((_N, _HID), jnp.float32),
    )(parts, parts, x0, wc_l)


def _tc_layer_final(parts, x0, wc_l, beta, w1t, b1):
    one_m_a = 1.0 - _ALPHA
    one_m_b = 1.0 - beta

    def body(pa_ref, pb_ref, x0_ref, w_ref, w1_ref, b1_ref, o_ref):
        agg = pa_ref[0] + pb_ref[0]
        hh = one_m_a * agg + _ALPHA * x0_ref[...]
        yw = jnp.dot(hh, w_ref[...], preferred_element_type=jnp.float32)
        hl = jnp.maximum(one_m_b * hh + beta * yw, 0.0)
        y = jnp.dot(hl, w1_ref[...], preferred_element_type=jnp.float32)
        y = y + b1_ref[...]
        m = jnp.max(y, axis=-1, keepdims=True)
        e = jnp.exp(y - m)
        lse = jnp.log(jnp.sum(e, axis=-1, keepdims=True))
        o_ref[...] = y - m - lse

    return pl.pallas_call(
        body,
        grid=(_GRID,),
        in_specs=[
            pl.BlockSpec((1, _ROWBLK, _HID), lambda i: (0, i, 0)),
            pl.BlockSpec((1, _ROWBLK, _HID), lambda i: (1, i, 0)),
            pl.BlockSpec((_ROWBLK, _HID), lambda i: (i, 0)),
            pl.BlockSpec((_HID, _HID), lambda i: (0, 0)),
            pl.BlockSpec((_HID, _OUT), lambda i: (0, 0)),
            pl.BlockSpec((1, _OUT), lambda i: (0, 0)),
        ],
        out_specs=pl.BlockSpec((_ROWBLK, _OUT), lambda i: (i, 0)),
        out_shape=jax.ShapeDtypeStruct((_N, _OUT), jnp.float32),
    )(parts, parts, x0, wc_l, w1t, b1)


def kernel(x, edge_index, W0, b0, W1, b1, Wc):
    ei = edge_index.astype(jnp.int32)
    npad = _EPAD - _E
    pad_src = jnp.arange(npad, dtype=jnp.int32) % 128
    pad_dst = _N + (jnp.arange(npad, dtype=jnp.int32) % 128)
    src3d = jnp.concatenate(
        [ei[0], pad_src]).reshape(_NCHUNKS, 1, _CHUNK)
    dst3d = jnp.concatenate(
        [ei[1], pad_dst]).reshape(_NCHUNKS, 1, _CHUNK)

    zrows = jnp.zeros((_ZROWS, _HID), jnp.float32)

    h = _tc_entry(x, W0.T, b0.reshape(1, _HID))
    x0 = h
    for l in range(_LAYERS - 1):
        parts = _sc_aggregate(h, src3d, dst3d, zrows)
        beta = float(np.log(_THETA / (l + 1) + 1.0))
        h = _tc_layer(parts, x0, Wc[l], beta)
    parts = _sc_aggregate(h, src3d, dst3d, zrows)
    beta = float(np.log(_THETA / _LAYERS + 1.0))
    return _tc_layer_final(parts, x0, Wc[_LAYERS - 1], beta,
                           W1.T, b1.reshape(1, _OUT))
